# baseline (speedup 1.0000x reference)
import jax
import jax.numpy as jnp
from jax.experimental import pallas as pl
from jax.experimental.pallas import tpu as pltpu

_BM = 512


def _gcn_block(adj_ref, x_ref, deg_ref, w_ref, b_ref, out_ref, acc_ref):
    i = pl.program_id(0)
    k = pl.program_id(1)
    h = x_ref.shape[0] // 2
    part = jnp.dot(adj_ref[...], x_ref[pl.ds(k * h, h), :],
                   preferred_element_type=jnp.float32)

    @pl.when(k == 0)
    def _():
        acc_ref[...] = part

    @pl.when(k == 1)
    def _():
        support = acc_ref[...] + part + x_ref[pl.ds(i * _BM, _BM), :]
        node_linear = jnp.dot(support, w_ref[...],
                              preferred_element_type=jnp.float32)
        out_ref[...] = node_linear / deg_ref[...] + b_ref[...]


def kernel(input, adj, node_degs, weight, bias):
    n, f_in = input.shape
    f_out = weight.shape[1]
    bm = _BM
    h = n // 2
    bias2 = bias.reshape(1, f_out)
    return pl.pallas_call(
        _gcn_block,
        grid=(n // bm, 2),
        in_specs=[
            pl.BlockSpec((bm, h), lambda i, k: (i, k)),
            pl.BlockSpec((n, f_in), lambda i, k: (0, 0)),
            pl.BlockSpec((bm, 1), lambda i, k: (i, 0)),
            pl.BlockSpec((f_in, f_out), lambda i, k: (0, 0)),
            pl.BlockSpec((1, f_out), lambda i, k: (0, 0)),
        ],
        out_specs=pl.BlockSpec((bm, f_out), lambda i, k: (i, 0)),
        out_shape=jax.ShapeDtypeStruct((n, f_out), jnp.float32),
        scratch_shapes=[pltpu.VMEM((bm, f_out), jnp.float32)],
        compiler_params=pltpu.CompilerParams(
            dimension_semantics=("parallel", "arbitrary"),
        ),
    )(adj, input, node_degs, weight, bias2)


# speedup vs baseline: 1.1990x; 1.1990x over previous
import jax
import jax.numpy as jnp
from jax.experimental import pallas as pl
from jax.experimental.pallas import tpu as pltpu

_BM = 512


def _gcn_block(adj_l_ref, adj_r_ref, x_ref, deg_ref, w_ref, b_ref, out_ref,
               sup_ref):
    i = pl.program_id(0)
    nstrips = pl.num_programs(0) - 1
    h = x_ref.shape[0] // 2

    @pl.when(i < nstrips)
    def _():
        acc = jnp.dot(adj_l_ref[...], x_ref[:h, :],
                      preferred_element_type=jnp.float32)
        acc += jnp.dot(adj_r_ref[...], x_ref[h:, :],
                       preferred_element_type=jnp.float32)
        sup_ref[i % 2] = acc + x_ref[pl.ds(i * _BM, _BM), :]

    @pl.when(i > 0)
    def _():
        node_linear = jnp.dot(sup_ref[(i + 1) % 2], w_ref[...],
                              preferred_element_type=jnp.float32)
        out_ref[...] = node_linear / deg_ref[...] + b_ref[...]


def kernel(input, adj, node_degs, weight, bias):
    n, f_in = input.shape
    f_out = weight.shape[1]
    bm = _BM
    h = n // 2
    nstrips = n // bm
    bias2 = bias.reshape(1, f_out)
    last = nstrips - 1
    return pl.pallas_call(
        _gcn_block,
        grid=(nstrips + 1,),
        in_specs=[
            pl.BlockSpec((bm, h), lambda i: (jnp.minimum(i, last), 0)),
            pl.BlockSpec((bm, h), lambda i: (jnp.minimum(i, last), 1)),
            pl.BlockSpec((n, f_in), lambda i: (0, 0)),
            pl.BlockSpec((bm, 1), lambda i: (jnp.maximum(i - 1, 0), 0)),
            pl.BlockSpec((f_in, f_out), lambda i: (0, 0)),
            pl.BlockSpec((1, f_out), lambda i: (0, 0)),
        ],
        out_specs=pl.BlockSpec((bm, f_out), lambda i: (jnp.maximum(i - 1, 0), 0)),
        out_shape=jax.ShapeDtypeStruct((n, f_out), jnp.float32),
        scratch_shapes=[pltpu.VMEM((2, bm, f_out), jnp.float32)],
        compiler_params=pltpu.CompilerParams(
            dimension_semantics=("arbitrary",),
        ),
    )(adj, adj, input, node_degs, weight, bias2)
